# Initial kernel scaffold; baseline (speedup 1.0000x reference)
import jax
import jax.numpy as jnp
from jax.experimental import pallas as pl


def kernel(x, batch):
    raise NotImplementedError("write your pallas kernel here")


# speedup vs baseline: 9.1850x; 9.1850x over previous
import functools

import jax
import jax.numpy as jnp
from jax import lax
from jax.experimental import pallas as pl
from jax.experimental.pallas import tpu as pltpu
from jax.experimental.pallas import tpu_sc as plsc

N_ROWS = 320000
N_FEAT = 128
N_SEG = 10000
BLK = 128
N_BLKS = N_ROWS // BLK
SLAB = 640
LAST_SLAB = N_SEG - 15 * SLAB

_mesh = plsc.VectorSubcoreMesh(core_axis_name="core", subcore_axis_name="subcore")


@functools.partial(
    pl.kernel,
    out_type=(
        jax.ShapeDtypeStruct((2, N_SEG, N_FEAT), jnp.float32),
        jax.ShapeDtypeStruct((N_SEG,), jnp.float32),
        jax.ShapeDtypeStruct((N_SEG,), jnp.float32),
    ),
    mesh=_mesh,
    scratch_types=[
        pltpu.VMEM_SHARED((N_SEG, N_FEAT), jnp.float32),
        pltpu.VMEM_SHARED((N_SEG,), jnp.float32),
        pltpu.VMEM((64, N_FEAT), jnp.float32),
        pltpu.VMEM((SLAB,), jnp.float32),
        pltpu.VMEM((BLK,), jnp.float32),
    ],
)
def _sc_segment_sum(x_hbm, b_hbm, sums_hbm, cnts0_hbm, cnts1_hbm,
                    sums_sh, cnts_sh, zbuf, zbuf1, ones_v):
    c = lax.axis_index("core")
    s = lax.axis_index("subcore")
    zero16 = jnp.zeros((16,), jnp.float32)
    one16 = jnp.ones((16,), jnp.float32)

    @pl.loop(0, 64)
    def _(r):
        for j in range(8):
            zbuf[r, pl.ds(j * 16, 16)] = zero16

    @pl.loop(0, SLAB // 16)
    def _(i):
        zbuf1[pl.ds(i * 16, 16)] = zero16

    for j in range(BLK // 16):
        ones_v[pl.ds(j * 16, 16)] = one16

    off = s * SLAB

    @pl.when(s < 15)
    def _():
        for k in range(SLAB // 64):
            pltpu.sync_copy(zbuf, sums_sh.at[pl.ds(off + k * 64, 64)])
        pltpu.sync_copy(zbuf1, cnts_sh.at[pl.ds(off, SLAB)])

    @pl.when(s == 15)
    def _():
        for k in range(LAST_SLAB // 64):
            pltpu.sync_copy(zbuf, sums_sh.at[pl.ds(off + k * 64, 64)])
        rem = LAST_SLAB % 64
        if rem:
            pltpu.sync_copy(zbuf.at[pl.ds(0, rem)],
                            sums_sh.at[pl.ds(off + LAST_SLAB - rem, rem)])
        pltpu.sync_copy(zbuf1.at[pl.ds(0, LAST_SLAB)],
                        cnts_sh.at[pl.ds(off, LAST_SLAB)])

    plsc.subcore_barrier()

    def scat_body(x_v, i_v):
        pltpu.sync_copy(x_v, sums_sh.at[i_v.at[0]], add=True)
        pltpu.sync_copy(ones_v, cnts_sh.at[i_v.at[0]], add=True)

    pltpu.emit_pipeline(
        scat_body,
        grid=(N_BLKS,),
        in_specs=[
            pl.BlockSpec((BLK, N_FEAT), lambda i: (i, 0)),
            pl.BlockSpec((1, BLK), lambda i: (i, 0)),
        ],
        core_axis_name=("core", "subcore"),
        dimension_semantics=(pltpu.PARALLEL,),
    )(x_hbm, b_hbm)

    plsc.subcore_barrier()

    for core_id, cnts_hbm in ((0, cnts0_hbm), (1, cnts1_hbm)):
        @pl.when((c == core_id) & (s < 15))
        def _():
            pltpu.sync_copy(sums_sh.at[pl.ds(off, SLAB)],
                            sums_hbm.at[c, pl.ds(off, SLAB)])
            pltpu.sync_copy(cnts_sh.at[pl.ds(off, SLAB)], zbuf1)
            pltpu.sync_copy(zbuf1, cnts_hbm.at[pl.ds(off, SLAB)])

        @pl.when((c == core_id) & (s == 15))
        def _():
            pltpu.sync_copy(sums_sh.at[pl.ds(off, LAST_SLAB)],
                            sums_hbm.at[c, pl.ds(off, LAST_SLAB)])
            pltpu.sync_copy(cnts_sh.at[pl.ds(off, LAST_SLAB)],
                            zbuf1.at[pl.ds(0, LAST_SLAB)])
            pltpu.sync_copy(zbuf1.at[pl.ds(0, LAST_SLAB)],
                            cnts_hbm.at[pl.ds(off, LAST_SLAB)])


def _div_body(s_ref, c0_ref, c1_ref, o_ref):
    sm = s_ref[0] + s_ref[1]
    ct = jnp.maximum(c0_ref[0, 0] + c1_ref[0, 0], 1.0)
    o_ref[...] = sm / ct[:, None]


_tc_divide = pl.pallas_call(
    _div_body,
    grid=(25,),
    in_specs=[
        pl.BlockSpec((2, 400, N_FEAT), lambda i: (0, i, 0)),
        pl.BlockSpec((1, 1, 400), lambda i: (i, 0, 0)),
        pl.BlockSpec((1, 1, 400), lambda i: (i, 0, 0)),
    ],
    out_specs=pl.BlockSpec((400, N_FEAT), lambda i: (i, 0)),
    out_shape=jax.ShapeDtypeStruct((N_SEG, N_FEAT), jnp.float32),
)


def kernel(x, batch):
    b32 = batch.astype(jnp.int32).reshape(N_BLKS, BLK)
    sums, cnts0, cnts1 = _sc_segment_sum(x, b32)
    return _tc_divide(sums,
                      cnts0.reshape(25, 1, 400),
                      cnts1.reshape(25, 1, 400))


# speedup vs baseline: 9.2103x; 1.0028x over previous
import functools

import jax
import jax.numpy as jnp
from jax import lax
from jax.experimental import pallas as pl
from jax.experimental.pallas import tpu as pltpu
from jax.experimental.pallas import tpu_sc as plsc

N_ROWS = 320000
N_FEAT = 128
N_SEG = 10000
BLK = 128
N_BLKS = N_ROWS // BLK
SLAB = 640
LAST_SLAB = N_SEG - 15 * SLAB

_mesh = plsc.VectorSubcoreMesh(core_axis_name="core", subcore_axis_name="subcore")


@functools.partial(
    pl.kernel,
    out_type=(
        jax.ShapeDtypeStruct((2, N_SEG, N_FEAT), jnp.float32),
        jax.ShapeDtypeStruct((N_SEG,), jnp.float32),
        jax.ShapeDtypeStruct((N_SEG,), jnp.float32),
    ),
    mesh=_mesh,
    scratch_types=[
        pltpu.VMEM_SHARED((N_SEG, N_FEAT), jnp.float32),
        pltpu.VMEM_SHARED((N_SEG,), jnp.float32),
        pltpu.VMEM((64, N_FEAT), jnp.float32),
        pltpu.VMEM((SLAB,), jnp.float32),
        pltpu.VMEM((BLK,), jnp.float32),
        pltpu.SemaphoreType.DMA,
    ],
)
def _sc_segment_sum(x_hbm, b_hbm, sums_hbm, cnts0_hbm, cnts1_hbm,
                    sums_sh, cnts_sh, zbuf, zbuf1, ones_v, scat_sem):
    c = lax.axis_index("core")
    s = lax.axis_index("subcore")
    zero16 = jnp.zeros((16,), jnp.float32)
    one16 = jnp.ones((16,), jnp.float32)

    @pl.loop(0, 64)
    def _(r):
        for j in range(8):
            zbuf[r, pl.ds(j * 16, 16)] = zero16

    @pl.loop(0, SLAB // 16)
    def _(i):
        zbuf1[pl.ds(i * 16, 16)] = zero16

    for j in range(BLK // 16):
        ones_v[pl.ds(j * 16, 16)] = one16

    off = s * SLAB

    @pl.when(s < 15)
    def _():
        for k in range(SLAB // 64):
            pltpu.sync_copy(zbuf, sums_sh.at[pl.ds(off + k * 64, 64)])
        pltpu.sync_copy(zbuf1, cnts_sh.at[pl.ds(off, SLAB)])

    @pl.when(s == 15)
    def _():
        for k in range(LAST_SLAB // 64):
            pltpu.sync_copy(zbuf, sums_sh.at[pl.ds(off + k * 64, 64)])
        rem = LAST_SLAB % 64
        if rem:
            pltpu.sync_copy(zbuf.at[pl.ds(0, rem)],
                            sums_sh.at[pl.ds(off + LAST_SLAB - rem, rem)])
        pltpu.sync_copy(zbuf1.at[pl.ds(0, LAST_SLAB)],
                        cnts_sh.at[pl.ds(off, LAST_SLAB)])

    plsc.subcore_barrier()

    def scat_body(x_v, i_v):
        a = pltpu.async_copy(x_v, sums_sh.at[i_v.at[0]], scat_sem, add=True)
        b = pltpu.async_copy(ones_v, cnts_sh.at[i_v.at[0]], scat_sem, add=True)
        a.wait()
        b.wait()

    pltpu.emit_pipeline(
        scat_body,
        grid=(N_BLKS,),
        in_specs=[
            pl.BlockSpec((BLK, N_FEAT), lambda i: (i, 0)),
            pl.BlockSpec((1, BLK), lambda i: (i, 0)),
        ],
        core_axis_name=("core", "subcore"),
        dimension_semantics=(pltpu.PARALLEL,),
        trace_scopes=False,
    )(x_hbm, b_hbm)

    plsc.subcore_barrier()

    for core_id, cnts_hbm in ((0, cnts0_hbm), (1, cnts1_hbm)):
        @pl.when((c == core_id) & (s < 15))
        def _():
            pltpu.sync_copy(sums_sh.at[pl.ds(off, SLAB)],
                            sums_hbm.at[c, pl.ds(off, SLAB)])
            pltpu.sync_copy(cnts_sh.at[pl.ds(off, SLAB)], zbuf1)
            pltpu.sync_copy(zbuf1, cnts_hbm.at[pl.ds(off, SLAB)])

        @pl.when((c == core_id) & (s == 15))
        def _():
            pltpu.sync_copy(sums_sh.at[pl.ds(off, LAST_SLAB)],
                            sums_hbm.at[c, pl.ds(off, LAST_SLAB)])
            pltpu.sync_copy(cnts_sh.at[pl.ds(off, LAST_SLAB)],
                            zbuf1.at[pl.ds(0, LAST_SLAB)])
            pltpu.sync_copy(zbuf1.at[pl.ds(0, LAST_SLAB)],
                            cnts_hbm.at[pl.ds(off, LAST_SLAB)])


def _div_body(s_ref, c0_ref, c1_ref, o_ref):
    sm = s_ref[0] + s_ref[1]
    ct = jnp.maximum(c0_ref[0, 0] + c1_ref[0, 0], 1.0)
    o_ref[...] = sm / ct[:, None]


_tc_divide = pl.pallas_call(
    _div_body,
    grid=(25,),
    in_specs=[
        pl.BlockSpec((2, 400, N_FEAT), lambda i: (0, i, 0)),
        pl.BlockSpec((1, 1, 400), lambda i: (i, 0, 0)),
        pl.BlockSpec((1, 1, 400), lambda i: (i, 0, 0)),
    ],
    out_specs=pl.BlockSpec((400, N_FEAT), lambda i: (i, 0)),
    out_shape=jax.ShapeDtypeStruct((N_SEG, N_FEAT), jnp.float32),
)


def kernel(x, batch):
    b32 = batch.astype(jnp.int32).reshape(N_BLKS, BLK)
    sums, cnts0, cnts1 = _sc_segment_sum(x, b32)
    return _tc_divide(sums,
                      cnts0.reshape(25, 1, 400),
                      cnts1.reshape(25, 1, 400))


# speedup vs baseline: 10.1381x; 1.1007x over previous
import functools

import jax
import jax.numpy as jnp
from jax import lax
from jax.experimental import pallas as pl
from jax.experimental.pallas import tpu as pltpu
from jax.experimental.pallas import tpu_sc as plsc

N_ROWS = 320000
N_FEAT = 128
N_SEG = 10000
BLK = 128
N_BLKS = N_ROWS // BLK
SLAB = 640
LAST_SLAB = N_SEG - 15 * SLAB

_mesh = plsc.VectorSubcoreMesh(core_axis_name="core", subcore_axis_name="subcore")


@functools.partial(
    pl.kernel,
    out_type=(
        jax.ShapeDtypeStruct((2, N_SEG, N_FEAT), jnp.float32),
        jax.ShapeDtypeStruct((N_SEG,), jnp.float32),
        jax.ShapeDtypeStruct((N_SEG,), jnp.float32),
    ),
    mesh=_mesh,
    scratch_types=[
        pltpu.VMEM_SHARED((N_SEG, N_FEAT), jnp.float32),
        pltpu.VMEM_SHARED((N_SEG,), jnp.float32),
        pltpu.VMEM((64, N_FEAT), jnp.float32),
        pltpu.VMEM((SLAB,), jnp.float32),
        pltpu.VMEM((BLK,), jnp.float32),
        pltpu.SemaphoreType.DMA,
    ],
)
def _sc_segment_sum(x_hbm, b_hbm, sums_hbm, cnts0_hbm, cnts1_hbm,
                    sums_sh, cnts_sh, zbuf, zbuf1, ones_v, scat_sem):
    c = lax.axis_index("core")
    s = lax.axis_index("subcore")
    zero16 = jnp.zeros((16,), jnp.float32)
    one16 = jnp.ones((16,), jnp.float32)

    @pl.loop(0, 64)
    def _(r):
        for j in range(8):
            zbuf[r, pl.ds(j * 16, 16)] = zero16

    @pl.loop(0, SLAB // 16)
    def _(i):
        zbuf1[pl.ds(i * 16, 16)] = zero16

    for j in range(BLK // 16):
        ones_v[pl.ds(j * 16, 16)] = one16

    off = s * SLAB

    @pl.when(s < 15)
    def _():
        for k in range(SLAB // 64):
            pltpu.sync_copy(zbuf, sums_sh.at[pl.ds(off + k * 64, 64)])
        pltpu.sync_copy(zbuf1, cnts_sh.at[pl.ds(off, SLAB)])

    @pl.when(s == 15)
    def _():
        for k in range(LAST_SLAB // 64):
            pltpu.sync_copy(zbuf, sums_sh.at[pl.ds(off + k * 64, 64)])
        rem = LAST_SLAB % 64
        if rem:
            pltpu.sync_copy(zbuf.at[pl.ds(0, rem)],
                            sums_sh.at[pl.ds(off + LAST_SLAB - rem, rem)])
        pltpu.sync_copy(zbuf1.at[pl.ds(0, LAST_SLAB)],
                        cnts_sh.at[pl.ds(off, LAST_SLAB)])

    plsc.subcore_barrier()

    def scat_body(x_v, i_v):
        a = pltpu.async_copy(x_v, sums_sh.at[i_v.at[0]], scat_sem, add=True)
        b = pltpu.async_copy(ones_v, cnts_sh.at[i_v.at[0]], scat_sem, add=True)
        a.wait()
        b.wait()

    pltpu.emit_pipeline(
        scat_body,
        grid=(N_BLKS,),
        in_specs=[
            pl.BlockSpec((BLK, N_FEAT), lambda i: (i, 0)),
            pl.BlockSpec((1, BLK), lambda i: (i, 0)),
        ],
        core_axis_name=("core", "subcore"),
        dimension_semantics=(pltpu.PARALLEL,),
        trace_scopes=False,
    )(x_hbm, b_hbm)

    plsc.subcore_barrier()

    for core_id, cnts_hbm in ((0, cnts0_hbm), (1, cnts1_hbm)):
        @pl.when((c == core_id) & (s < 15))
        def _():
            pltpu.sync_copy(sums_sh.at[pl.ds(off, SLAB)],
                            sums_hbm.at[c, pl.ds(off, SLAB)])
            pltpu.sync_copy(cnts_sh.at[pl.ds(off, SLAB)], zbuf1)
            pltpu.sync_copy(zbuf1, cnts_hbm.at[pl.ds(off, SLAB)])

        @pl.when((c == core_id) & (s == 15))
        def _():
            pltpu.sync_copy(sums_sh.at[pl.ds(off, LAST_SLAB)],
                            sums_hbm.at[c, pl.ds(off, LAST_SLAB)])
            pltpu.sync_copy(cnts_sh.at[pl.ds(off, LAST_SLAB)],
                            zbuf1.at[pl.ds(0, LAST_SLAB)])
            pltpu.sync_copy(zbuf1.at[pl.ds(0, LAST_SLAB)],
                            cnts_hbm.at[pl.ds(off, LAST_SLAB)])


def _div_body(s_ref, c0_ref, c1_ref, o_ref):
    sm = s_ref[0] + s_ref[1]
    ct = jnp.maximum(c0_ref[...] + c1_ref[...], 1.0)
    o_ref[...] = sm / ct[:, None]


_tc_divide = pl.pallas_call(
    _div_body,
    out_shape=jax.ShapeDtypeStruct((N_SEG, N_FEAT), jnp.float32),
)


def kernel(x, batch):
    b32 = batch.astype(jnp.int32).reshape(N_BLKS, BLK)
    sums, cnts0, cnts1 = _sc_segment_sum(x, b32)
    return _tc_divide(sums, cnts0, cnts1)
